# baseline (speedup 1.0000x reference)
import jax
import jax.numpy as jnp
from jax.experimental import pallas as pl
from jax.experimental.pallas import tpu as pltpu

_J = 100
_M = 28
_NN = _J + _M
_H = 5
_EMBED = 64
_DIMS = (16, 64, _EMBED)
_WID = _H * _NN
_G = 4


def _bd_wide(out):
    k = jax.lax.broadcasted_iota(jnp.int32, (_H * out, _WID), 0)
    c = jax.lax.broadcasted_iota(jnp.int32, (_H * out, _WID), 1)
    return jnp.where((k // out) == (c // _NN), 1.0, 0.0).astype(jnp.float32)


def _bd_head(out):
    k = jax.lax.broadcasted_iota(jnp.int32, (_H * out, _H), 0)
    h = jax.lax.broadcasted_iota(jnp.int32, (_H * out, _H), 1)
    return jnp.where((k // out) == h, 1.0, 0.0).astype(jnp.float32)


def _bd_head_t(out):
    h = jax.lax.broadcasted_iota(jnp.int32, (_H, _H * out), 0)
    k = jax.lax.broadcasted_iota(jnp.int32, (_H, _H * out), 1)
    return jnp.where((k // out) == h, 1.0, 0.0).astype(jnp.float32)


def _layer(ft, m5, ef5, el_rhs, bdh_s, ce_s, Wef, bf, out):
    el_part = jnp.concatenate(
        [jnp.dot(ft[:, h * out:(h + 1) * out], el_rhs[h],
                 preferred_element_type=jnp.float32)
         for h in range(_H)], axis=1)
    er_t = jax.lax.dot_general(
        bdh_s, ft, (((0,), (1,)), ((), ())),
        preferred_element_type=jnp.float32)
    er_s = jnp.concatenate([er_t[h:h + 1, :] for h in range(_H)], axis=1)
    e = el_part + er_s + ef5 * ce_s
    e = jnp.maximum(e, 0.2 * e)
    e_m = jnp.where(m5, e, -3.4e38)
    emax = jnp.max(e_m, axis=0, keepdims=True)
    emax = jnp.where(emax > -1e37, emax, 0.0)
    ex = jnp.exp(e_m - emax)
    den = jnp.sum(ex, axis=0, keepdims=True)
    a = ex * (1.0 / jnp.where(den > 0, den, 1.0))
    aef = a * ef5
    acc = jnp.zeros((_NN, out), jnp.float32)
    for h in range(_H):
        sl = slice(h * _NN, (h + 1) * _NN)
        ft_h = ft[:, h * out:(h + 1) * out]
        rst = jax.lax.dot_general(
            a[:, sl], ft_h, (((0,), (0,)), ((), ())),
            preferred_element_type=jnp.float32)
        we_b = jnp.broadcast_to(Wef[0:1, h * out:(h + 1) * out], (_NN, out))
        rst = rst + jax.lax.dot_general(
            aef[:, sl], we_b, (((0,), (0,)), ((), ())),
            preferred_element_type=jnp.float32)
        rst = rst + bf[0:1, h * out:(h + 1) * out]
        acc = acc + jnp.maximum(rst, 0.01 * rst)
    return acc * (1.0 / _H)


def _body(graph_ref, t_ref, nh_ref, nl_ref, nw_ref, np_ref, nn_ref,
          W0_ref, alc0_ref, arc0_ref, We0_ref, aef0_ref, b0_ref,
          W1_ref, alc1_ref, arc1_ref, We1_ref, aef1_ref, b1_ref,
          W2_ref, alc2_ref, arc2_ref, We2_ref, aef2_ref, b2_ref,
          out_ref):
    def _prep(alc, arc, Wef, aefw, out):
        el_rhs = [jnp.broadcast_to(alc[h * out:(h + 1) * out], (out, _NN))
                  for h in range(_H)]
        bdh_s = _bd_head(out) * arc
        ce5 = jnp.dot(Wef * aefw, _bd_head(out),
                      preferred_element_type=jnp.float32)
        ce_s = jnp.concatenate(
            [jnp.broadcast_to(ce5[0:1, h:h + 1], (1, _NN)) for h in range(_H)],
            axis=1)
        return el_rhs, bdh_s, ce_s

    elr0, bdh0, ces0 = _prep(alc0_ref[...], arc0_ref[...], We0_ref[...],
                             aef0_ref[...], _DIMS[0])
    elr1, bdh1, ces1 = _prep(alc1_ref[...], arc1_ref[...], We1_ref[...],
                             aef1_ref[...], _DIMS[1])
    elr2, bdh2, ces2 = _prep(alc2_ref[...], arc2_ref[...], We2_ref[...],
                             aef2_ref[...], _DIMS[2])
    zrow = jnp.zeros((_M, _NN), jnp.float32)
    zcol = jnp.zeros((_J, _M), jnp.float32)
    z28 = jnp.zeros((1, _M), jnp.float32)
    z100 = jnp.zeros((1, _J), jnp.float32)
    lane = jax.lax.broadcasted_iota(jnp.int32, (1, _NN), 1)
    jid = jnp.where(lane < _J, (lane + 1).astype(jnp.float32), 0.0)
    mid = jnp.where(lane >= _J, (lane - (_J - 1)).astype(jnp.float32), 0.0)
    zline = jnp.zeros((1, _NN), jnp.float32)
    for g in range(_G):
        g_pad = jnp.concatenate([graph_ref[g], zrow], axis=0)
        ef = jnp.concatenate(
            [jnp.concatenate([t_ref[g], zcol], axis=1), zrow], axis=0)
        m5 = jnp.concatenate([g_pad] * _H, axis=1) != 0.0
        ef5 = jnp.concatenate([ef] * _H, axis=1)
        r0 = jnp.concatenate([nh_ref[g], z28], axis=1)
        r1 = jnp.concatenate([nl_ref[g], z28], axis=1)
        r2 = jnp.concatenate([z100, jnp.broadcast_to(nw_ref[g], (1, _M))],
                             axis=1)
        r3 = jnp.concatenate([z100, jnp.broadcast_to(np_ref[g], (1, _M))],
                             axis=1)
        r4 = jnp.concatenate([z100, jnp.broadcast_to(nn_ref[g], (1, _M))],
                             axis=1)
        x_t = jnp.concatenate(
            [r0, r1, r2, r3, r4, jid, mid, zline], axis=0)
        ft = jax.lax.dot_general(
            x_t, _pad8(W0_ref[...]), (((0,), (0,)), ((), ())),
            preferred_element_type=jnp.float32)
        x = _layer(ft, m5, ef5, elr0, bdh0, ces0, We0_ref[...], b0_ref[...],
                   _DIMS[0])
        ft = jnp.dot(x, W1_ref[...], preferred_element_type=jnp.float32)
        x = _layer(ft, m5, ef5, elr1, bdh1, ces1, We1_ref[...], b1_ref[...],
                   _DIMS[1])
        ft = jnp.dot(x, W2_ref[...], preferred_element_type=jnp.float32)
        x = _layer(ft, m5, ef5, elr2, bdh2, ces2, We2_ref[...], b2_ref[...],
                   _DIMS[2])
        out_ref[g] = x


def _pad8(W):
    return jnp.concatenate([W, jnp.zeros((1, W.shape[1]), jnp.float32)],
                           axis=0)


def kernel(Graph, norm_h, norm_L, norm_W, norm_P, norm_N, T,
           W0, We0, al0, ar0, ae0, b0,
           W1, We1, al1, ar1, ae1, b1,
           W2, We2, al2, ar2, ae2, b2):
    bs = Graph.shape[0]

    nh3 = norm_h.reshape(bs, 1, _J)
    nl3 = norm_L.reshape(bs, 1, _J)
    nw3 = norm_W.reshape(bs, 1, 1)
    np3 = norm_P.reshape(bs, 1, 1)
    nn3 = norm_N.reshape(bs, 1, 1)

    alf0, arf0, aef0 = al0.reshape(-1, 1), ar0.reshape(-1, 1), ae0.reshape(1, -1)
    alf1, arf1, aef1 = al1.reshape(-1, 1), ar1.reshape(-1, 1), ae1.reshape(1, -1)
    alf2, arf2, aef2 = al2.reshape(-1, 1), ar2.reshape(-1, 1), ae2.reshape(1, -1)
    b0r, b1r, b2r = b0.reshape(1, -1), b1.reshape(1, -1), b2.reshape(1, -1)

    def bspec(shape3):
        return pl.BlockSpec((_G,) + shape3[1:], lambda b: (b, 0, 0))

    def wspec(arr):
        return pl.BlockSpec(arr.shape, lambda b: (0,) * arr.ndim)

    out = pl.pallas_call(
        _body,
        grid=(bs // _G,),
        in_specs=[
            bspec(Graph.shape), bspec(T.shape),
            bspec(nh3.shape), bspec(nl3.shape), bspec(nw3.shape),
            bspec(np3.shape), bspec(nn3.shape),
            wspec(W0), wspec(alf0), wspec(arf0), wspec(We0), wspec(aef0),
            wspec(b0r),
            wspec(W1), wspec(alf1), wspec(arf1), wspec(We1), wspec(aef1),
            wspec(b1r),
            wspec(W2), wspec(alf2), wspec(arf2), wspec(We2), wspec(aef2),
            wspec(b2r),
        ],
        out_specs=pl.BlockSpec((_G, _NN, _EMBED), lambda b: (b, 0, 0)),
        out_shape=jax.ShapeDtypeStruct((bs, _NN, _EMBED), jnp.float32),
        compiler_params=pltpu.CompilerParams(
            dimension_semantics=("parallel",)),
    )(Graph, T, nh3, nl3, nw3, np3, nn3,
      W0, alf0, arf0, We0, aef0, b0r,
      W1, alf1, arf1, We1, aef1, b1r,
      W2, alf2, arf2, We2, aef2, b2r)
    return out


# speedup vs baseline: 1.2497x; 1.2497x over previous
import jax
import jax.numpy as jnp
from jax.experimental import pallas as pl
from jax.experimental.pallas import tpu as pltpu

_J = 100
_M = 28
_NN = _J + _M
_H = 5
_EMBED = 64
_DIMS = (16, 64, _EMBED)
_WID = _H * _NN
_G = 4


def _bd_wide(out):
    k = jax.lax.broadcasted_iota(jnp.int32, (_H * out, _WID), 0)
    c = jax.lax.broadcasted_iota(jnp.int32, (_H * out, _WID), 1)
    return jnp.where((k // out) == (c // _NN), 1.0, 0.0).astype(jnp.float32)


def _bd_head(out):
    k = jax.lax.broadcasted_iota(jnp.int32, (_H * out, _H), 0)
    h = jax.lax.broadcasted_iota(jnp.int32, (_H * out, _H), 1)
    return jnp.where((k // out) == h, 1.0, 0.0).astype(jnp.float32)


def _bd_head_t(out):
    h = jax.lax.broadcasted_iota(jnp.int32, (_H, _H * out), 0)
    k = jax.lax.broadcasted_iota(jnp.int32, (_H, _H * out), 1)
    return jnp.where((k // out) == h, 1.0, 0.0).astype(jnp.float32)


def _layer(ft, m5, ef5, bdw, bdh, alf, arf, Wef, aef_w, bf, out):
    el_part = jnp.dot(ft * alf, bdw,
                      preferred_element_type=jnp.float32)
    er_t = jax.lax.dot_general(
        bdh, ft * arf, (((0,), (1,)), ((), ())),
        preferred_element_type=jnp.float32)
    er_s = jnp.concatenate([er_t[h:h + 1, :] for h in range(_H)], axis=1)
    ce_s = jnp.dot(Wef * aef_w, bdw,
                   preferred_element_type=jnp.float32)
    e = el_part + er_s + ef5 * ce_s
    e = jnp.maximum(e, 0.2 * e)
    e_m = jnp.where(m5, e, -3.4e38)
    emax = jnp.max(e_m, axis=0, keepdims=True)
    emax = jnp.where(emax > -1e37, emax, 0.0)
    ex = jnp.exp(e_m - emax)
    den = jnp.sum(ex, axis=0, keepdims=True)
    a = ex * (1.0 / jnp.where(den > 0, den, 1.0))
    aef = a * ef5
    acc = jnp.zeros((_NN, out), jnp.float32)
    for h in range(_H):
        sl = slice(h * _NN, (h + 1) * _NN)
        ft_h = ft[:, h * out:(h + 1) * out]
        rst = jax.lax.dot_general(
            a[:, sl], ft_h, (((0,), (0,)), ((), ())),
            preferred_element_type=jnp.float32)
        we_b = jnp.broadcast_to(Wef[0:1, h * out:(h + 1) * out], (_NN, out))
        rst = rst + jax.lax.dot_general(
            aef[:, sl], we_b, (((0,), (0,)), ((), ())),
            preferred_element_type=jnp.float32)
        rst = rst + bf[0:1, h * out:(h + 1) * out]
        acc = acc + jnp.maximum(rst, 0.01 * rst)
    return acc * (1.0 / _H)


def _body(graph_ref, t_ref, nh_ref, nl_ref, nw_ref, np_ref, nn_ref,
          W0_ref, alc0_ref, arc0_ref, We0_ref, aef0_ref, b0_ref,
          W1_ref, alc1_ref, arc1_ref, We1_ref, aef1_ref, b1_ref,
          W2_ref, alc2_ref, arc2_ref, We2_ref, aef2_ref, b2_ref,
          out_ref):
    bd16 = _bd_wide(_DIMS[0])
    bd64 = _bd_wide(_DIMS[1])
    bdh16 = _bd_head(_DIMS[0])
    bdh64 = _bd_head(_DIMS[1])
    zrow = jnp.zeros((_M, _NN), jnp.float32)
    zcol = jnp.zeros((_J, _M), jnp.float32)
    z28 = jnp.zeros((1, _M), jnp.float32)
    z100 = jnp.zeros((1, _J), jnp.float32)
    lane = jax.lax.broadcasted_iota(jnp.int32, (1, _NN), 1)
    jid = jnp.where(lane < _J, (lane + 1).astype(jnp.float32), 0.0)
    mid = jnp.where(lane >= _J, (lane - (_J - 1)).astype(jnp.float32), 0.0)
    zline = jnp.zeros((1, _NN), jnp.float32)
    for g in range(_G):
        g_pad = jnp.concatenate([graph_ref[g], zrow], axis=0)
        ef = jnp.concatenate(
            [jnp.concatenate([t_ref[g], zcol], axis=1), zrow], axis=0)
        m5 = jnp.concatenate([g_pad] * _H, axis=1) != 0.0
        ef5 = jnp.concatenate([ef] * _H, axis=1)
        r0 = jnp.concatenate([nh_ref[g], z28], axis=1)
        r1 = jnp.concatenate([nl_ref[g], z28], axis=1)
        r2 = jnp.concatenate([z100, jnp.broadcast_to(nw_ref[g], (1, _M))],
                             axis=1)
        r3 = jnp.concatenate([z100, jnp.broadcast_to(np_ref[g], (1, _M))],
                             axis=1)
        r4 = jnp.concatenate([z100, jnp.broadcast_to(nn_ref[g], (1, _M))],
                             axis=1)
        x_t = jnp.concatenate(
            [r0, r1, r2, r3, r4, jid, mid, zline], axis=0)
        ft = jax.lax.dot_general(
            x_t, _pad8(W0_ref[...]), (((0,), (0,)), ((), ())),
            preferred_element_type=jnp.float32)
        x = _layer(ft, m5, ef5, bd16, bdh16, alc0_ref[...], arc0_ref[...],
                   We0_ref[...], aef0_ref[...], b0_ref[...], _DIMS[0])
        ft = jnp.dot(x, W1_ref[...], preferred_element_type=jnp.float32)
        x = _layer(ft, m5, ef5, bd64, bdh64, alc1_ref[...], arc1_ref[...],
                   We1_ref[...], aef1_ref[...], b1_ref[...], _DIMS[1])
        ft = jnp.dot(x, W2_ref[...], preferred_element_type=jnp.float32)
        x = _layer(ft, m5, ef5, bd64, bdh64, alc2_ref[...], arc2_ref[...],
                   We2_ref[...], aef2_ref[...], b2_ref[...], _DIMS[2])
        out_ref[g] = x


def _pad8(W):
    return jnp.concatenate([W, jnp.zeros((1, W.shape[1]), jnp.float32)],
                           axis=0)


def kernel(Graph, norm_h, norm_L, norm_W, norm_P, norm_N, T,
           W0, We0, al0, ar0, ae0, b0,
           W1, We1, al1, ar1, ae1, b1,
           W2, We2, al2, ar2, ae2, b2):
    bs = Graph.shape[0]

    nh3 = norm_h.reshape(bs, 1, _J)
    nl3 = norm_L.reshape(bs, 1, _J)
    nw3 = norm_W.reshape(bs, 1, 1)
    np3 = norm_P.reshape(bs, 1, 1)
    nn3 = norm_N.reshape(bs, 1, 1)

    alf0, arf0, aef0 = al0.reshape(1, -1), ar0.reshape(1, -1), ae0.reshape(1, -1)
    alf1, arf1, aef1 = al1.reshape(1, -1), ar1.reshape(1, -1), ae1.reshape(1, -1)
    alf2, arf2, aef2 = al2.reshape(1, -1), ar2.reshape(1, -1), ae2.reshape(1, -1)
    b0r, b1r, b2r = b0.reshape(1, -1), b1.reshape(1, -1), b2.reshape(1, -1)

    def bspec(shape3):
        return pl.BlockSpec((_G,) + shape3[1:], lambda b: (b, 0, 0))

    def wspec(arr):
        return pl.BlockSpec(arr.shape, lambda b: (0,) * arr.ndim)

    out = pl.pallas_call(
        _body,
        grid=(bs // _G,),
        in_specs=[
            bspec(Graph.shape), bspec(T.shape),
            bspec(nh3.shape), bspec(nl3.shape), bspec(nw3.shape),
            bspec(np3.shape), bspec(nn3.shape),
            wspec(W0), wspec(alf0), wspec(arf0), wspec(We0), wspec(aef0),
            wspec(b0r),
            wspec(W1), wspec(alf1), wspec(arf1), wspec(We1), wspec(aef1),
            wspec(b1r),
            wspec(W2), wspec(alf2), wspec(arf2), wspec(We2), wspec(aef2),
            wspec(b2r),
        ],
        out_specs=pl.BlockSpec((_G, _NN, _EMBED), lambda b: (b, 0, 0)),
        out_shape=jax.ShapeDtypeStruct((bs, _NN, _EMBED), jnp.float32),
        compiler_params=pltpu.CompilerParams(
            dimension_semantics=("parallel",)),
    )(Graph, T, nh3, nl3, nw3, np3, nn3,
      W0, alf0, arf0, We0, aef0, b0r,
      W1, alf1, arf1, We1, aef1, b1r,
      W2, alf2, arf2, We2, aef2, b2r)
    return out
